# baseline (speedup 1.0000x reference)
import functools

import jax
import jax.numpy as jnp
from jax import lax
from jax.experimental import pallas as pl
from jax.experimental.pallas import tpu as pltpu
from jax.experimental.pallas import tpu_sc as plsc

BATCH = 16384
SEQ = 50
DIM = 128
B = BATCH * SEQ

CHUNK = 128
NBUF = 5
LOOK = 2


@functools.lru_cache(maxsize=None)
def _build():
    info = plsc.get_sparse_core_info()
    nc, ns = info.num_cores, info.num_subcores
    nw = nc * ns
    b_per_w = B // nw
    chunks = b_per_w // CHUNK

    mesh = plsc.VectorSubcoreMesh(core_axis_name="c", subcore_axis_name="s")

    @functools.partial(
        pl.kernel,
        out_type=jax.ShapeDtypeStruct((B, DIM), jnp.float32),
        mesh=mesh,
        scratch_types=[
            pltpu.VMEM((chunks, CHUNK), jnp.int32),
            [pltpu.VMEM((CHUNK, DIM), jnp.float32)] * NBUF,
            [pltpu.SemaphoreType.DMA] * NBUF,
            [pltpu.SemaphoreType.DMA] * NBUF,
        ],
    )
    def emb_kernel(idx_hbm, table_hbm, out_hbm, idx_v, rows, gsems, wsems):
        wid = lax.axis_index("s") * nc + lax.axis_index("c")
        pltpu.sync_copy(idx_hbm.at[pl.ds(wid * chunks, chunks)], idx_v)

        row0 = wid * b_per_w

        def start_gather(g, b):
            pltpu.async_copy(table_hbm.at[idx_v.at[g]], rows[b], gsems[b])

        def wait_gather(g, b):
            pltpu.make_async_copy(table_hbm.at[idx_v.at[g]], rows[b],
                                  gsems[b]).wait()

        def start_wb(g, b):
            pltpu.async_copy(rows[b],
                             out_hbm.at[pl.ds(row0 + g * CHUNK, CHUNK)],
                             wsems[b])

        def wait_wb(g, b):
            pltpu.make_async_copy(rows[b],
                                  out_hbm.at[pl.ds(row0 + g * CHUNK, CHUNK)],
                                  wsems[b]).wait()

        for g in range(LOOK):
            start_gather(g, g % NBUF)

        for g in range(NBUF - LOOK):
            start_gather(g + LOOK, (g + LOOK) % NBUF)
            wait_gather(g, g % NBUF)
            start_wb(g, g % NBUF)

        def body(outer):
            for d in range(NBUF):
                g = outer + d
                bf = (NBUF - LOOK + LOOK + d) % NBUF
                b = (NBUF - LOOK + d) % NBUF
                wait_wb(g + LOOK - NBUF, bf)
                start_gather(g + LOOK, bf)
                wait_gather(g, b)
                start_wb(g, b)

        pl.loop(NBUF - LOOK, chunks - LOOK, step=NBUF)(body)

        for g in range(chunks - LOOK, chunks):
            wait_gather(g, g % NBUF)
            start_wb(g, g % NBUF)

        for g in range(chunks - NBUF, chunks):
            wait_wb(g, g % NBUF)

    return emb_kernel


def kernel(x, table):
    idx = jnp.swapaxes(x, 0, 1).reshape(B // CHUNK, CHUNK)
    out = _build()(idx, table)
    return jnp.swapaxes(out.reshape(SEQ, BATCH, DIM), 0, 1)


# speedup vs baseline: 1.0022x; 1.0022x over previous
import functools

import jax
import jax.numpy as jnp
from jax import lax
from jax.experimental import pallas as pl
from jax.experimental.pallas import tpu as pltpu
from jax.experimental.pallas import tpu_sc as plsc

BATCH = 16384
SEQ = 50
DIM = 128
B = BATCH * SEQ

CHUNK = 128
NBUF = 5
LOOK = 3


@functools.lru_cache(maxsize=None)
def _build():
    info = plsc.get_sparse_core_info()
    nc, ns = info.num_cores, info.num_subcores
    nw = nc * ns
    b_per_w = B // nw
    chunks = b_per_w // CHUNK
    assert (chunks - NBUF) % NBUF == 0 and LOOK < NBUF

    mesh = plsc.VectorSubcoreMesh(core_axis_name="c", subcore_axis_name="s")

    @functools.partial(
        pl.kernel,
        out_type=jax.ShapeDtypeStruct((B // CHUNK, CHUNK, DIM), jnp.float32),
        mesh=mesh,
        scratch_types=[
            pltpu.VMEM((chunks, CHUNK), jnp.int32),
            [pltpu.VMEM((CHUNK, DIM), jnp.float32)] * NBUF,
            [pltpu.SemaphoreType.DMA] * NBUF,
            [pltpu.SemaphoreType.DMA] * NBUF,
        ],
    )
    def emb_kernel(idx_hbm, table_hbm, out_hbm, idx_v, rows, gsems, wsems):
        wid = lax.axis_index("s") * nc + lax.axis_index("c")
        pltpu.sync_copy(idx_hbm.at[pl.ds(wid * chunks, chunks)], idx_v)

        def start_gather(g, b):
            pltpu.async_copy(table_hbm.at[idx_v.at[g]], rows[b], gsems[b])

        def wait_gather(g, b):
            pltpu.make_async_copy(table_hbm.at[idx_v.at[g]], rows[b],
                                  gsems[b]).wait()

        def start_wb(g, b):
            pltpu.async_copy(rows[b], out_hbm.at[wid * chunks + g], wsems[b])

        def wait_wb(g, b):
            pltpu.make_async_copy(rows[b], out_hbm.at[wid * chunks + g],
                                  wsems[b]).wait()

        for g in range(LOOK):
            start_gather(g, g % NBUF)

        for g in range(NBUF - LOOK):
            start_gather(g + LOOK, (g + LOOK) % NBUF)
            wait_gather(g, g % NBUF)
            start_wb(g, g % NBUF)

        def body(outer):
            for d in range(NBUF):
                g = outer + d
                bf = (NBUF - LOOK + LOOK + d) % NBUF
                b = (NBUF - LOOK + d) % NBUF
                wait_wb(g + LOOK - NBUF, bf)
                start_gather(g + LOOK, bf)
                wait_gather(g, b)
                start_wb(g, b)

        pl.loop(NBUF - LOOK, chunks - LOOK, step=NBUF)(body)

        for g in range(chunks - LOOK, chunks):
            wait_gather(g, g % NBUF)
            start_wb(g, g % NBUF)

        for g in range(chunks - NBUF, chunks):
            wait_wb(g, g % NBUF)

    return emb_kernel


def kernel(x, table):
    idx = jnp.swapaxes(x, 0, 1).reshape(B // CHUNK, CHUNK)
    out = _build()(idx, table)
    return jnp.swapaxes(out.reshape(SEQ, BATCH, DIM), 0, 1)
